# baseline (device time: 114229 ns/iter reference)
import jax
import jax.numpy as jnp
from jax import lax
from jax.experimental import pallas as pl
from jax.experimental.pallas import tpu as pltpu

N_DEV = 16
B, S, C_IN, C_OUT = 4, 1024, 512, 512
ROWS = B * S
HALF = ROWS // 2
CHUNK = HALF // N_DEV
CPB = S // CHUNK
N_HOPS = N_DEV - 1

RING = [0, 4, 8, 12, 15, 11, 7, 3, 2, 6, 10, 14, 13, 9, 5, 1]
POS = [0] * N_DEV
for _i, _l in enumerate(RING):
    POS[_l] = _i
RIGHT = [RING[(POS[l] + 1) % N_DEV] for l in range(N_DEV)]
LEFT = [RING[(POS[l] - 1) % N_DEV] for l in range(N_DEV)]


def _lut(table, idx):
    acc = jnp.int32(table[0])
    for i in range(1, len(table)):
        acc = jnp.where(idx == i, jnp.int32(table[i]), acc)
    return acc


def kernel(x, k, Wp):
    def body(x_ref, k_ref, w_ref, out_ref, cw_sb, ccw_sb, cw_stage, ccw_stage,
             cw_ag, ccw_ag,
             cw_rs_send, cw_rs_recv, cw_ag_send, cw_ag_recv,
             ccw_rs_send, ccw_rs_recv, ccw_ag_send, ccw_ag_recv):
        my = lax.axis_index("i")
        p = _lut(POS, my)
        right = _lut(RIGHT, my)
        left = _lut(LEFT, my)

        barrier = pltpu.get_barrier_semaphore()
        pl.semaphore_signal(barrier, inc=1, device_id=(left,),
                            device_id_type=pl.DeviceIdType.MESH)
        pl.semaphore_signal(barrier, inc=1, device_id=(right,),
                            device_id_type=pl.DeviceIdType.MESH)
        pl.semaphore_wait(barrier, 2)

        kv = k_ref[:, :]

        def compute_chunk(c, b_base, half_base):
            b = lax.div(c, CPB) + b_base
            s0 = pl.multiple_of(lax.rem(c, CPB) * CHUNK, CHUNK)
            xc = x_ref[b, pl.ds(s0, CHUNK), :]
            hs = pl.multiple_of(jnp.maximum(s0 - 8, 0), 8)
            halo = x_ref[b, pl.ds(hs, 8), :][5:8]
            halo = jnp.where(s0 == 0, jnp.zeros_like(halo), halo)
            xe = jnp.concatenate([halo, xc], axis=0)
            accv = xe[3:3 + CHUNK] * kv[3][None, :]
            for t in range(3):
                accv = accv + xe[t:t + CHUNK] * kv[t][None, :]
            av = accv / (1.0 + jnp.exp(-accv))
            out_ref[pl.ds(half_base + c * CHUNK, CHUNK), :] = (
                jax.lax.dot_general(
                    av, w_ref[:, :], (((1,), (0,)), ((), ())),
                    preferred_element_type=jnp.float32,
                )
            )

        def chunk_id(i):
            return lax.rem(p + i + 2 * N_DEV, N_DEV)

        def rd(src, dst, send_sem, recv_sem, dev):
            return pltpu.make_async_remote_copy(
                src_ref=src, dst_ref=dst, send_sem=send_sem,
                recv_sem=recv_sem, device_id=(dev,),
                device_id_type=pl.DeviceIdType.MESH,
            )

        def out_chunk(off):
            return out_ref[pl.ds(off, CHUNK), :]

        xv = x_ref[:, :, :]
        accf = xv * kv[3][None, None, :]
        for t in range(3):
            sh = 3 - t
            shiftedf = jnp.concatenate(
                [jnp.zeros((B, sh, C_IN), jnp.float32), xv[:, : S - sh, :]],
                axis=1,
            )
            accf = accf + shiftedf * kv[t][None, None, :]
        af = accf / (1.0 + jnp.exp(-accf))
        out_ref[:, :] = jax.lax.dot_general(
            af.reshape(ROWS, C_IN), w_ref[:, :],
            (((1,), (0,)), ((), ())),
            preferred_element_type=jnp.float32,
        )

        pending_sends = []

        cw_sb[0, :, :] = out_chunk(chunk_id(0) * CHUNK).astype(jnp.bfloat16)
        ccw_sb[0, :, :] = out_chunk(
            HALF + chunk_id(0) * CHUNK).astype(jnp.bfloat16)

        for s in range(N_HOPS):
            cw = rd(cw_sb.at[s], cw_stage.at[s],
                    cw_rs_send.at[s], cw_rs_recv.at[s], right)
            ccw = rd(ccw_sb.at[s], ccw_stage.at[s],
                     ccw_rs_send.at[s], ccw_rs_recv.at[s], left)
            cw.start()
            ccw.start()
            pending_sends += [cw, ccw]
            cw.wait_recv()
            off = chunk_id(-s - 1) * CHUNK
            summed = out_chunk(off) + cw_stage[s, :, :].astype(jnp.float32)
            out_ref[pl.ds(off, CHUNK), :] = summed
            if s < N_HOPS - 1:
                cw_sb[s + 1, :, :] = summed.astype(jnp.bfloat16)
            else:
                cw_ag[pl.ds(chunk_id(1) * CHUNK, CHUNK), :] = (
                    summed.astype(jnp.bfloat16))
            ccw.wait_recv()
            off = HALF + chunk_id(s + 1) * CHUNK
            summed = out_chunk(off) + ccw_stage[s, :, :].astype(jnp.float32)
            out_ref[pl.ds(off, CHUNK), :] = summed
            if s < N_HOPS - 1:
                ccw_sb[s + 1, :, :] = summed.astype(jnp.bfloat16)
            else:
                ccw_ag[pl.ds(chunk_id(-1) * CHUNK, CHUNK), :] = (
                    summed.astype(jnp.bfloat16))

        for s in range(N_HOPS):
            cw_off = chunk_id(1 - s) * CHUNK
            ccw_off = chunk_id(s - 1) * CHUNK
            cw = rd(cw_ag.at[pl.ds(cw_off, CHUNK), :],
                    cw_ag.at[pl.ds(cw_off, CHUNK), :],
                    cw_ag_send.at[s], cw_ag_recv.at[s], right)
            ccw = rd(ccw_ag.at[pl.ds(ccw_off, CHUNK), :],
                     ccw_ag.at[pl.ds(ccw_off, CHUNK), :],
                     ccw_ag_send.at[s], ccw_ag_recv.at[s], left)
            cw.start()
            ccw.start()
            pending_sends += [cw, ccw]
            if s > 0:
                o = chunk_id(1 - s) * CHUNK
                out_ref[pl.ds(o, CHUNK), :] = (
                    cw_ag[pl.ds(o, CHUNK), :].astype(jnp.float32))
                o = chunk_id(s - 1) * CHUNK
                out_ref[pl.ds(HALF + o, CHUNK), :] = (
                    ccw_ag[pl.ds(o, CHUNK), :].astype(jnp.float32))
            cw.wait_recv()
            ccw.wait_recv()
        o = chunk_id(-N_HOPS + 1) * CHUNK
        out_ref[pl.ds(o, CHUNK), :] = (
            cw_ag[pl.ds(o, CHUNK), :].astype(jnp.float32))
        o = chunk_id(N_HOPS - 1) * CHUNK
        out_ref[pl.ds(HALF + o, CHUNK), :] = (
            ccw_ag[pl.ds(o, CHUNK), :].astype(jnp.float32))

        for r in pending_sends:
            r.wait_send()

    out = pl.pallas_call(
        body,
        out_shape=jax.ShapeDtypeStruct((ROWS, C_OUT), jnp.float32),
        in_specs=[pl.BlockSpec(memory_space=pltpu.VMEM)] * 3,
        out_specs=pl.BlockSpec(memory_space=pltpu.VMEM),
        scratch_shapes=[
            pltpu.VMEM((N_HOPS, CHUNK, C_OUT), jnp.bfloat16),
            pltpu.VMEM((N_HOPS, CHUNK, C_OUT), jnp.bfloat16),
            pltpu.VMEM((N_HOPS, CHUNK, C_OUT), jnp.bfloat16),
            pltpu.VMEM((N_HOPS, CHUNK, C_OUT), jnp.bfloat16),
            pltpu.VMEM((HALF, C_OUT), jnp.bfloat16),
            pltpu.VMEM((HALF, C_OUT), jnp.bfloat16),
        ] + [pltpu.SemaphoreType.DMA((N_HOPS,))] * 8,
        compiler_params=pltpu.CompilerParams(collective_id=0),
    )(x, k, Wp)
    return out.reshape(B, S, C_OUT)


# device time: 109316 ns/iter; 1.0449x vs baseline; 1.0449x over previous
import jax
import jax.numpy as jnp
from jax import lax
from jax.experimental import pallas as pl
from jax.experimental.pallas import tpu as pltpu

N_DEV = 16
B, S, C_IN, C_OUT = 4, 1024, 512, 512
ROWS = B * S
HALF = ROWS // 2
CHUNK = HALF // N_DEV
CPB = S // CHUNK
N_HOPS = N_DEV - 1

RING = [0, 4, 8, 12, 15, 11, 7, 3, 2, 6, 10, 14, 13, 9, 5, 1]
POS = [0] * N_DEV
for _i, _l in enumerate(RING):
    POS[_l] = _i
RIGHT = [RING[(POS[l] + 1) % N_DEV] for l in range(N_DEV)]
LEFT = [RING[(POS[l] - 1) % N_DEV] for l in range(N_DEV)]


def _lut(table, idx):
    acc = jnp.int32(table[0])
    for i in range(1, len(table)):
        acc = jnp.where(idx == i, jnp.int32(table[i]), acc)
    return acc


def kernel(x, k, Wp):
    def body(x_ref, k_ref, w_ref, out_ref, cw_sb, ccw_sb, cw_stage, ccw_stage,
             cw_ag, ccw_ag,
             cw_rs_send, cw_rs_recv, cw_ag_send, cw_ag_recv,
             ccw_rs_send, ccw_rs_recv, ccw_ag_send, ccw_ag_recv):
        my = lax.axis_index("i")
        p = _lut(POS, my)
        right = _lut(RIGHT, my)
        left = _lut(LEFT, my)

        barrier = pltpu.get_barrier_semaphore()
        pl.semaphore_signal(barrier, inc=1, device_id=(left,),
                            device_id_type=pl.DeviceIdType.MESH)
        pl.semaphore_signal(barrier, inc=1, device_id=(right,),
                            device_id_type=pl.DeviceIdType.MESH)
        pl.semaphore_wait(barrier, 2)

        kv = k_ref[:, :]
        wv_bf = w_ref[:, :].astype(jnp.bfloat16)

        def compute_chunk(c, b_base, half_base):
            b = lax.div(c, CPB) + b_base
            s0 = pl.multiple_of(lax.rem(c, CPB) * CHUNK, CHUNK)
            xc = x_ref[b, pl.ds(s0, CHUNK), :]
            hs = pl.multiple_of(jnp.maximum(s0 - 8, 0), 8)
            halo = x_ref[b, pl.ds(hs, 8), :][5:8]
            halo = jnp.where(s0 == 0, jnp.zeros_like(halo), halo)
            xe = jnp.concatenate([halo, xc], axis=0)
            accv = xe[3:3 + CHUNK] * kv[3][None, :]
            for t in range(3):
                accv = accv + xe[t:t + CHUNK] * kv[t][None, :]
            av = accv / (1.0 + jnp.exp(-accv))
            out_ref[pl.ds(half_base + c * CHUNK, CHUNK), :] = (
                jax.lax.dot_general(
                    av.astype(jnp.bfloat16), wv_bf, (((1,), (0,)), ((), ())),
                    preferred_element_type=jnp.float32,
                )
            )

        def chunk_id(i):
            return lax.rem(p + i + 2 * N_DEV, N_DEV)

        def rd(src, dst, send_sem, recv_sem, dev):
            return pltpu.make_async_remote_copy(
                src_ref=src, dst_ref=dst, send_sem=send_sem,
                recv_sem=recv_sem, device_id=(dev,),
                device_id_type=pl.DeviceIdType.MESH,
            )

        def out_chunk(off):
            return out_ref[pl.ds(off, CHUNK), :]

        compute_chunk(chunk_id(0), 0, 0)
        compute_chunk(chunk_id(0), 2, HALF)
        compute_chunk(chunk_id(-1), 0, 0)
        compute_chunk(chunk_id(1), 2, HALF)

        pending_sends = []

        cw_sb[0, :, :] = out_chunk(chunk_id(0) * CHUNK).astype(jnp.bfloat16)
        ccw_sb[0, :, :] = out_chunk(
            HALF + chunk_id(0) * CHUNK).astype(jnp.bfloat16)

        for s in range(N_HOPS):
            cw = rd(cw_sb.at[s], cw_stage.at[s],
                    cw_rs_send.at[s], cw_rs_recv.at[s], right)
            ccw = rd(ccw_sb.at[s], ccw_stage.at[s],
                     ccw_rs_send.at[s], ccw_rs_recv.at[s], left)
            cw.start()
            ccw.start()
            pending_sends += [cw, ccw]
            if s < N_HOPS - 1:
                compute_chunk(chunk_id(-s - 2), 0, 0)
                compute_chunk(chunk_id(s + 2), 2, HALF)
            cw.wait_recv()
            off = chunk_id(-s - 1) * CHUNK
            summed = out_chunk(off) + cw_stage[s, :, :].astype(jnp.float32)
            out_ref[pl.ds(off, CHUNK), :] = summed
            if s < N_HOPS - 1:
                cw_sb[s + 1, :, :] = summed.astype(jnp.bfloat16)
            else:
                cw_ag[pl.ds(chunk_id(1) * CHUNK, CHUNK), :] = (
                    summed.astype(jnp.bfloat16))
            ccw.wait_recv()
            off = HALF + chunk_id(s + 1) * CHUNK
            summed = out_chunk(off) + ccw_stage[s, :, :].astype(jnp.float32)
            out_ref[pl.ds(off, CHUNK), :] = summed
            if s < N_HOPS - 1:
                ccw_sb[s + 1, :, :] = summed.astype(jnp.bfloat16)
            else:
                ccw_ag[pl.ds(chunk_id(-1) * CHUNK, CHUNK), :] = (
                    summed.astype(jnp.bfloat16))

        for s in range(N_HOPS):
            cw_off = chunk_id(1 - s) * CHUNK
            ccw_off = chunk_id(s - 1) * CHUNK
            cw = rd(cw_ag.at[pl.ds(cw_off, CHUNK), :],
                    cw_ag.at[pl.ds(cw_off, CHUNK), :],
                    cw_ag_send.at[s], cw_ag_recv.at[s], right)
            ccw = rd(ccw_ag.at[pl.ds(ccw_off, CHUNK), :],
                     ccw_ag.at[pl.ds(ccw_off, CHUNK), :],
                     ccw_ag_send.at[s], ccw_ag_recv.at[s], left)
            cw.start()
            ccw.start()
            pending_sends += [cw, ccw]
            if s > 0:
                o = chunk_id(1 - s) * CHUNK
                out_ref[pl.ds(o, CHUNK), :] = (
                    cw_ag[pl.ds(o, CHUNK), :].astype(jnp.float32))
                o = chunk_id(s - 1) * CHUNK
                out_ref[pl.ds(HALF + o, CHUNK), :] = (
                    ccw_ag[pl.ds(o, CHUNK), :].astype(jnp.float32))
            cw.wait_recv()
            ccw.wait_recv()
        o = chunk_id(-N_HOPS + 1) * CHUNK
        out_ref[pl.ds(o, CHUNK), :] = (
            cw_ag[pl.ds(o, CHUNK), :].astype(jnp.float32))
        o = chunk_id(N_HOPS - 1) * CHUNK
        out_ref[pl.ds(HALF + o, CHUNK), :] = (
            ccw_ag[pl.ds(o, CHUNK), :].astype(jnp.float32))

        for r in pending_sends:
            r.wait_send()

    out = pl.pallas_call(
        body,
        out_shape=jax.ShapeDtypeStruct((ROWS, C_OUT), jnp.float32),
        in_specs=[pl.BlockSpec(memory_space=pltpu.VMEM)] * 3,
        out_specs=pl.BlockSpec(memory_space=pltpu.VMEM),
        scratch_shapes=[
            pltpu.VMEM((N_HOPS, CHUNK, C_OUT), jnp.bfloat16),
            pltpu.VMEM((N_HOPS, CHUNK, C_OUT), jnp.bfloat16),
            pltpu.VMEM((N_HOPS, CHUNK, C_OUT), jnp.bfloat16),
            pltpu.VMEM((N_HOPS, CHUNK, C_OUT), jnp.bfloat16),
            pltpu.VMEM((HALF, C_OUT), jnp.bfloat16),
            pltpu.VMEM((HALF, C_OUT), jnp.bfloat16),
        ] + [pltpu.SemaphoreType.DMA((N_HOPS,))] * 8,
        compiler_params=pltpu.CompilerParams(collective_id=0),
    )(x, k, Wp)
    return out.reshape(B, S, C_OUT)


# device time: 62636 ns/iter; 1.8237x vs baseline; 1.7453x over previous
import os

import jax
import jax.numpy as jnp
from jax import lax
from jax.experimental import pallas as pl
from jax.experimental.pallas import tpu as pltpu

ABLATE = int(os.environ.get("ABLATE", "0"))
DO_RS = ABLATE < 2
DO_AG = ABLATE < 1

N_DEV = 16
B, S, C_IN, C_OUT = 4, 1024, 512, 512
ROWS = B * S
HALF = ROWS // 2
CHUNK = HALF // N_DEV
CPB = S // CHUNK
N_HOPS = N_DEV - 1

RING = [0, 4, 8, 12, 15, 11, 7, 3, 2, 6, 10, 14, 13, 9, 5, 1]
POS = [0] * N_DEV
for _i, _l in enumerate(RING):
    POS[_l] = _i
RIGHT = [RING[(POS[l] + 1) % N_DEV] for l in range(N_DEV)]
LEFT = [RING[(POS[l] - 1) % N_DEV] for l in range(N_DEV)]


def _lut(table, idx):
    acc = jnp.int32(table[0])
    for i in range(1, len(table)):
        acc = jnp.where(idx == i, jnp.int32(table[i]), acc)
    return acc


def kernel(x, k, Wp):
    def body(x_ref, k_ref, w_ref, out_ref, cw_sb, ccw_sb, cw_stage, ccw_stage,
             cw_ag, ccw_ag,
             cw_rs_send, cw_rs_recv, cw_ag_send, cw_ag_recv,
             ccw_rs_send, ccw_rs_recv, ccw_ag_send, ccw_ag_recv):
        my = lax.axis_index("i")
        p = _lut(POS, my)
        right = _lut(RIGHT, my)
        left = _lut(LEFT, my)

        barrier = pltpu.get_barrier_semaphore()
        pl.semaphore_signal(barrier, inc=1, device_id=(left,),
                            device_id_type=pl.DeviceIdType.MESH)
        pl.semaphore_signal(barrier, inc=1, device_id=(right,),
                            device_id_type=pl.DeviceIdType.MESH)
        pl.semaphore_wait(barrier, 2)

        kv = k_ref[:, :]
        wv_bf = w_ref[:, :].astype(jnp.bfloat16)

        def compute_chunk(c, b_base, half_base):
            b = lax.div(c, CPB) + b_base
            s0 = pl.multiple_of(lax.rem(c, CPB) * CHUNK, CHUNK)
            xc = x_ref[b, pl.ds(s0, CHUNK), :]
            hs = pl.multiple_of(jnp.maximum(s0 - 8, 0), 8)
            halo = x_ref[b, pl.ds(hs, 8), :][5:8]
            halo = jnp.where(s0 == 0, jnp.zeros_like(halo), halo)
            xe = jnp.concatenate([halo, xc], axis=0)
            accv = xe[3:3 + CHUNK] * kv[3][None, :]
            for t in range(3):
                accv = accv + xe[t:t + CHUNK] * kv[t][None, :]
            av = accv / (1.0 + jnp.exp(-accv))
            out_ref[pl.ds(half_base + c * CHUNK, CHUNK), :] = (
                jax.lax.dot_general(
                    av.astype(jnp.bfloat16), wv_bf, (((1,), (0,)), ((), ())),
                    preferred_element_type=jnp.float32,
                )
            )

        def chunk_id(i):
            return lax.rem(p + i + 2 * N_DEV, N_DEV)

        def rd(src, dst, send_sem, recv_sem, dev):
            return pltpu.make_async_remote_copy(
                src_ref=src, dst_ref=dst, send_sem=send_sem,
                recv_sem=recv_sem, device_id=(dev,),
                device_id_type=pl.DeviceIdType.MESH,
            )

        def out_chunk(off):
            return out_ref[pl.ds(off, CHUNK), :]

        compute_chunk(chunk_id(0), 0, 0)
        compute_chunk(chunk_id(0), 2, HALF)
        compute_chunk(chunk_id(-1), 0, 0)
        compute_chunk(chunk_id(1), 2, HALF)

        pending_sends = []

        cw_sb[0, :, :] = out_chunk(chunk_id(0) * CHUNK).astype(jnp.bfloat16)
        ccw_sb[0, :, :] = out_chunk(
            HALF + chunk_id(0) * CHUNK).astype(jnp.bfloat16)

        for s in range(N_HOPS):
            if DO_RS:
                cw = rd(cw_sb.at[s], cw_stage.at[s],
                        cw_rs_send.at[s], cw_rs_recv.at[s], right)
                ccw = rd(ccw_sb.at[s], ccw_stage.at[s],
                         ccw_rs_send.at[s], ccw_rs_recv.at[s], left)
                cw.start()
                ccw.start()
                pending_sends += [cw, ccw]
            if s < N_HOPS - 1:
                compute_chunk(chunk_id(-s - 2), 0, 0)
                compute_chunk(chunk_id(s + 2), 2, HALF)
            if not DO_RS:
                continue
            cw.wait_recv()
            off = chunk_id(-s - 1) * CHUNK
            summed = out_chunk(off) + cw_stage[s, :, :].astype(jnp.float32)
            out_ref[pl.ds(off, CHUNK), :] = summed
            if s < N_HOPS - 1:
                cw_sb[s + 1, :, :] = summed.astype(jnp.bfloat16)
            else:
                cw_ag[pl.ds(chunk_id(1) * CHUNK, CHUNK), :] = (
                    summed.astype(jnp.bfloat16))
            ccw.wait_recv()
            off = HALF + chunk_id(s + 1) * CHUNK
            summed = out_chunk(off) + ccw_stage[s, :, :].astype(jnp.float32)
            out_ref[pl.ds(off, CHUNK), :] = summed
            if s < N_HOPS - 1:
                ccw_sb[s + 1, :, :] = summed.astype(jnp.bfloat16)
            else:
                ccw_ag[pl.ds(chunk_id(-1) * CHUNK, CHUNK), :] = (
                    summed.astype(jnp.bfloat16))

        for s in range(N_HOPS if DO_AG else 0):
            cw_off = chunk_id(1 - s) * CHUNK
            ccw_off = chunk_id(s - 1) * CHUNK
            cw = rd(cw_ag.at[pl.ds(cw_off, CHUNK), :],
                    cw_ag.at[pl.ds(cw_off, CHUNK), :],
                    cw_ag_send.at[s], cw_ag_recv.at[s], right)
            ccw = rd(ccw_ag.at[pl.ds(ccw_off, CHUNK), :],
                     ccw_ag.at[pl.ds(ccw_off, CHUNK), :],
                     ccw_ag_send.at[s], ccw_ag_recv.at[s], left)
            cw.start()
            ccw.start()
            pending_sends += [cw, ccw]
            if s > 0:
                o = chunk_id(1 - s) * CHUNK
                out_ref[pl.ds(o, CHUNK), :] = (
                    cw_ag[pl.ds(o, CHUNK), :].astype(jnp.float32))
                o = chunk_id(s - 1) * CHUNK
                out_ref[pl.ds(HALF + o, CHUNK), :] = (
                    ccw_ag[pl.ds(o, CHUNK), :].astype(jnp.float32))
            cw.wait_recv()
            ccw.wait_recv()
        if DO_AG:
            o = chunk_id(-N_HOPS + 1) * CHUNK
            out_ref[pl.ds(o, CHUNK), :] = (
                cw_ag[pl.ds(o, CHUNK), :].astype(jnp.float32))
            o = chunk_id(N_HOPS - 1) * CHUNK
            out_ref[pl.ds(HALF + o, CHUNK), :] = (
                ccw_ag[pl.ds(o, CHUNK), :].astype(jnp.float32))

        for r in pending_sends:
            r.wait_send()

    out = pl.pallas_call(
        body,
        out_shape=jax.ShapeDtypeStruct((ROWS, C_OUT), jnp.float32),
        in_specs=[pl.BlockSpec(memory_space=pltpu.VMEM)] * 3,
        out_specs=pl.BlockSpec(memory_space=pltpu.VMEM),
        scratch_shapes=[
            pltpu.VMEM((N_HOPS, CHUNK, C_OUT), jnp.bfloat16),
            pltpu.VMEM((N_HOPS, CHUNK, C_OUT), jnp.bfloat16),
            pltpu.VMEM((N_HOPS, CHUNK, C_OUT), jnp.bfloat16),
            pltpu.VMEM((N_HOPS, CHUNK, C_OUT), jnp.bfloat16),
            pltpu.VMEM((HALF, C_OUT), jnp.bfloat16),
            pltpu.VMEM((HALF, C_OUT), jnp.bfloat16),
        ] + [pltpu.SemaphoreType.DMA((N_HOPS,))] * 8,
        compiler_params=pltpu.CompilerParams(collective_id=0),
    )(x, k, Wp)
    return out.reshape(B, S, C_OUT)
